# baseline (speedup 1.0000x reference)
import functools

import jax
import jax.numpy as jnp
from jax import lax
from jax.experimental import pallas as pl
from jax.experimental.pallas import tpu as pltpu
from jax.experimental.pallas import tpu_sc as plsc

N = 10000
E = 320000
D = 128
R = 8
RP = R + 1

NC = 2
NS = 16
L = 16
NW = NC * NS
EPW = E // NW
C = 80
KCH = EPW // C
NBUF = 3
NP = 10240
RPT = NP // NS

BN = 2000

_GDN = lax.GatherDimensionNumbers(
    offset_dims=(), collapsed_slice_dims=(0,), start_index_map=(0,))


def _lane_bcast(vec, i):
    idx = jnp.full((L, 1), i, jnp.int32)
    return lax.gather(vec, idx, _GDN, (1,),
                      mode=lax.GatherScatterMode.PROMISE_IN_BOUNDS)


def _proj_body(h_ref, w_ref, out_ref):
    out_ref[0] = jnp.dot(h_ref[...], w_ref[0], preferred_element_type=jnp.float32)


_proj_call = pl.pallas_call(
    _proj_body,
    grid=(N // BN, RP),
    in_specs=[
        pl.BlockSpec((BN, D), lambda i, j: (i, 0)),
        pl.BlockSpec((1, D, D), lambda i, j: (j, 0, 0)),
    ],
    out_specs=pl.BlockSpec((1, BN, D), lambda i, j: (j, i, 0)),
    out_shape=jax.ShapeDtypeStruct((RP, N, D), jnp.float32),
)


def _combine_body(parts_ref, pself_ref, b_ref, out_ref):
    x = parts_ref[0] + parts_ref[1] + pself_ref[0] + b_ref[...]
    out_ref[...] = jnp.maximum(x, 0.0)


_combine_call = pl.pallas_call(
    _combine_body,
    grid=(N // BN,),
    in_specs=[
        pl.BlockSpec((2, BN, D), lambda i: (0, i, 0)),
        pl.BlockSpec((1, BN, D), lambda i: (R, i, 0)),
        pl.BlockSpec((1, D), lambda i: (0, 0)),
    ],
    out_specs=pl.BlockSpec((BN, D), lambda i: (i, 0)),
    out_shape=jax.ShapeDtypeStruct((N, D), jnp.float32),
)


@functools.partial(
    pl.kernel,
    mesh=plsc.VectorSubcoreMesh(core_axis_name="c", subcore_axis_name="s"),
    out_type=jax.ShapeDtypeStruct((2, NP, D), jnp.float32),
    scratch_types=[
        [pltpu.VMEM((C,), jnp.int32) for _ in range(NBUF)],
        [pltpu.VMEM((C,), jnp.int32) for _ in range(NBUF)],
        [pltpu.VMEM((C,), jnp.int32) for _ in range(NBUF)],
        [pltpu.VMEM((C,), jnp.float32) for _ in range(NBUF)],
        [pltpu.VMEM((C,), jnp.int32) for _ in range(NBUF)],
        [pltpu.VMEM((C, D), jnp.float32) for _ in range(NBUF)],
        pltpu.VMEM_SHARED((NP, D), jnp.float32),
        [pltpu.SemaphoreType.DMA for _ in range(NBUF)],
        [pltpu.SemaphoreType.DMA for _ in range(NBUF)],
        [pltpu.SemaphoreType.DMA for _ in range(NBUF)],
    ],
)
def _mp_call(proj_hbm, src_hbm, dst_hbm, r_hbm, norm_hbm, zeros_hbm, out_hbm,
             srcc, relc, gidxc, normc, dstc, rows, agg_sh, sem_m, sem_g, sem_s):
    c = lax.axis_index("c")
    s = lax.axis_index("s")
    wid = s * NC + c
    ebase = wid * EPW

    pltpu.sync_copy(zeros_hbm.at[pl.ds(s * RPT, RPT)],
                    agg_sh.at[pl.ds(s * RPT, RPT)])
    plsc.subcore_barrier()

    def meta_start(k, b):
        base = ebase + k * C
        pltpu.async_copy(src_hbm.at[pl.ds(base, C)], srcc[b], sem_m[b])
        pltpu.async_copy(r_hbm.at[pl.ds(base, C)], relc[b], sem_m[b])
        pltpu.async_copy(norm_hbm.at[pl.ds(base, C)], normc[b], sem_m[b])
        pltpu.async_copy(dst_hbm.at[pl.ds(base, C)], dstc[b], sem_m[b])

    def meta_wait_build_gather(b):
        pltpu.make_async_copy(src_hbm.at[pl.ds(0, C)], srcc[b], sem_m[b]).wait()
        pltpu.make_async_copy(r_hbm.at[pl.ds(0, C)], relc[b], sem_m[b]).wait()
        pltpu.make_async_copy(norm_hbm.at[pl.ds(0, C)], normc[b], sem_m[b]).wait()
        pltpu.make_async_copy(dst_hbm.at[pl.ds(0, C)], dstc[b], sem_m[b]).wait()
        for t in range(C // L):
            sl = pl.ds(t * L, L)
            gidxc[b][sl] = relc[b][sl] * N + srcc[b][sl]
        pltpu.async_copy(proj_hbm.at[gidxc[b]], rows[b], sem_g[b])

    def gather_wait(b):
        pltpu.make_async_copy(proj_hbm.at[gidxc[b]], rows[b], sem_g[b]).wait()

    def scale(b):
        def _e16(t, cy):
            eb = t * L
            nv = normc[b][pl.ds(eb, L)]
            for i in range(L):
                nb = _lane_bcast(nv, i)
                for j in range(D // L):
                    sl = pl.ds(j * L, L)
                    rows[b][eb + i, sl] = rows[b][eb + i, sl] * nb
            return cy

        lax.fori_loop(0, C // L, _e16, 0)

    def scat_start(b):
        pltpu.async_copy(rows[b], agg_sh.at[dstc[b]], sem_s[b], add=True)

    def scat_wait(b):
        pltpu.make_async_copy(rows[b], agg_sh.at[dstc[b]], sem_s[b]).wait()

    meta_start(0, 0)
    meta_start(1, 1)
    meta_wait_build_gather(0)

    def _group(g, carry):
        for b in range(NBUF):
            k = 3 * g + b
            prev = (b + 2) % NBUF

            @pl.when(k < KCH)
            def _():
                @pl.when(k + 1 < KCH)
                def _():
                    meta_wait_build_gather((b + 1) % NBUF)

                gather_wait(b)
                scale(b)
                scat_start(b)

                @pl.when(k >= 1)
                def _():
                    scat_wait(prev)

                @pl.when(k + 2 < KCH)
                def _():
                    meta_start(k + 2, prev)

        return carry

    lax.fori_loop(0, (KCH + NBUF - 1) // NBUF, _group, 0)
    scat_wait((KCH - 1) % NBUF)

    plsc.subcore_barrier()
    pltpu.sync_copy(agg_sh.at[pl.ds(s * RPT, RPT)],
                    out_hbm.at[c, pl.ds(s * RPT, RPT)])


def _layer(h, src, dst, r, norm1, zeros, W, Wself, b):
    Wall = jnp.concatenate([W, Wself[None]], axis=0)
    proj = _proj_call(h, Wall)
    parts = _mp_call(proj.reshape(RP * N, D), src, dst, r, norm1, zeros)
    return _combine_call(parts, proj, b.reshape(1, D))


def kernel(h, edge_index, r, norm, W1, Wself1, b1, W2, Wself2, b2):
    norm1 = norm.reshape(E)
    src = edge_index[0]
    dst = edge_index[1]
    zeros = jnp.zeros((NP, D), jnp.float32)
    h1 = _layer(h, src, dst, r, norm1, zeros, W1, Wself1, b1)
    h2 = _layer(h1, src, dst, r, norm1, zeros, W2, Wself2, b2)
    return h2


# speedup vs baseline: 1.0273x; 1.0273x over previous
import functools

import jax
import jax.numpy as jnp
from jax import lax
from jax.experimental import pallas as pl
from jax.experimental.pallas import tpu as pltpu
from jax.experimental.pallas import tpu_sc as plsc

N = 10000
E = 320000
D = 128
R = 8
RP = R + 1

NC = 2
NS = 16
L = 16
NW = NC * NS
EPW = E // NW
C = 80
KCH = EPW // C
NBUF = 4
NP = 10240
RPT = NP // NS

BN = 2000

_GDN = lax.GatherDimensionNumbers(
    offset_dims=(), collapsed_slice_dims=(0,), start_index_map=(0,))


def _lane_bcast(vec, i):
    idx = jnp.full((L, 1), i, jnp.int32)
    return lax.gather(vec, idx, _GDN, (1,),
                      mode=lax.GatherScatterMode.PROMISE_IN_BOUNDS)


def _proj_body(h_ref, w_ref, out_ref):
    out_ref[0] = jnp.dot(h_ref[...], w_ref[0], preferred_element_type=jnp.float32)


_proj_call = pl.pallas_call(
    _proj_body,
    grid=(N // BN, RP),
    in_specs=[
        pl.BlockSpec((BN, D), lambda i, j: (i, 0)),
        pl.BlockSpec((1, D, D), lambda i, j: (j, 0, 0)),
    ],
    out_specs=pl.BlockSpec((1, BN, D), lambda i, j: (j, i, 0)),
    out_shape=jax.ShapeDtypeStruct((RP, N, D), jnp.float32),
)


def _combine_body(parts_ref, pself_ref, b_ref, out_ref):
    x = parts_ref[0] + parts_ref[1] + pself_ref[0] + b_ref[...]
    out_ref[...] = jnp.maximum(x, 0.0)


_combine_call = pl.pallas_call(
    _combine_body,
    grid=(N // BN,),
    in_specs=[
        pl.BlockSpec((2, BN, D), lambda i: (0, i, 0)),
        pl.BlockSpec((1, BN, D), lambda i: (R, i, 0)),
        pl.BlockSpec((1, D), lambda i: (0, 0)),
    ],
    out_specs=pl.BlockSpec((BN, D), lambda i: (i, 0)),
    out_shape=jax.ShapeDtypeStruct((N, D), jnp.float32),
)


@functools.partial(
    pl.kernel,
    mesh=plsc.VectorSubcoreMesh(core_axis_name="c", subcore_axis_name="s"),
    out_type=jax.ShapeDtypeStruct((2, NP, D), jnp.float32),
    scratch_types=[
        [pltpu.VMEM((C,), jnp.int32) for _ in range(NBUF)],
        [pltpu.VMEM((C,), jnp.int32) for _ in range(NBUF)],
        [pltpu.VMEM((C,), jnp.int32) for _ in range(NBUF)],
        [pltpu.VMEM((C,), jnp.float32) for _ in range(NBUF)],
        [pltpu.VMEM((C,), jnp.int32) for _ in range(NBUF)],
        [pltpu.VMEM((C, D), jnp.float32) for _ in range(NBUF)],
        pltpu.VMEM_SHARED((NP, D), jnp.float32),
        [pltpu.SemaphoreType.DMA for _ in range(NBUF)],
        [pltpu.SemaphoreType.DMA for _ in range(NBUF)],
        [pltpu.SemaphoreType.DMA for _ in range(NBUF)],
    ],
)
def _mp_call(proj_hbm, src_hbm, dst_hbm, r_hbm, norm_hbm, zeros_hbm, out_hbm,
             srcc, relc, gidxc, normc, dstc, rows, agg_sh, sem_m, sem_g, sem_s):
    c = lax.axis_index("c")
    s = lax.axis_index("s")
    wid = s * NC + c
    ebase = wid * EPW

    pltpu.sync_copy(zeros_hbm.at[pl.ds(s * RPT, RPT)],
                    agg_sh.at[pl.ds(s * RPT, RPT)])
    plsc.subcore_barrier()

    def meta_start(k, b):
        base = ebase + k * C
        pltpu.async_copy(src_hbm.at[pl.ds(base, C)], srcc[b], sem_m[b])
        pltpu.async_copy(r_hbm.at[pl.ds(base, C)], relc[b], sem_m[b])
        pltpu.async_copy(norm_hbm.at[pl.ds(base, C)], normc[b], sem_m[b])
        pltpu.async_copy(dst_hbm.at[pl.ds(base, C)], dstc[b], sem_m[b])

    def meta_wait_build_gather(b):
        pltpu.make_async_copy(src_hbm.at[pl.ds(0, C)], srcc[b], sem_m[b]).wait()
        pltpu.make_async_copy(r_hbm.at[pl.ds(0, C)], relc[b], sem_m[b]).wait()
        pltpu.make_async_copy(norm_hbm.at[pl.ds(0, C)], normc[b], sem_m[b]).wait()
        pltpu.make_async_copy(dst_hbm.at[pl.ds(0, C)], dstc[b], sem_m[b]).wait()
        for t in range(C // L):
            sl = pl.ds(t * L, L)
            gidxc[b][sl] = relc[b][sl] * N + srcc[b][sl]
        pltpu.async_copy(proj_hbm.at[gidxc[b]], rows[b], sem_g[b])

    def gather_wait(b):
        pltpu.make_async_copy(proj_hbm.at[gidxc[b]], rows[b], sem_g[b]).wait()

    def scale(b):
        def _e16(t, cy):
            eb = t * L
            nv = normc[b][pl.ds(eb, L)]
            for i in range(L):
                nb = _lane_bcast(nv, i)
                for j in range(D // L):
                    sl = pl.ds(j * L, L)
                    rows[b][eb + i, sl] = rows[b][eb + i, sl] * nb
            return cy

        lax.fori_loop(0, C // L, _e16, 0)

    def scat_start(b):
        pltpu.async_copy(rows[b], agg_sh.at[dstc[b]], sem_s[b], add=True)

    def scat_wait(b):
        pltpu.make_async_copy(rows[b], agg_sh.at[dstc[b]], sem_s[b]).wait()

    meta_start(0, 0)
    meta_start(1, 1)
    meta_start(2, 2)
    meta_wait_build_gather(0)
    meta_wait_build_gather(1)

    def _group(g, carry):
        for b in range(NBUF):
            k = NBUF * g + b

            @pl.when(k < KCH)
            def _():
                @pl.when(k + 2 < KCH)
                def _():
                    meta_wait_build_gather((b + 2) % NBUF)

                gather_wait(b)
                scale(b)
                scat_start(b)

                @pl.when(k >= 1)
                def _():
                    scat_wait((b + 3) % NBUF)

                @pl.when(k + 3 < KCH)
                def _():
                    meta_start(k + 3, (b + 3) % NBUF)

        return carry

    lax.fori_loop(0, (KCH + NBUF - 1) // NBUF, _group, 0)
    scat_wait((KCH - 1) % NBUF)

    plsc.subcore_barrier()
    pltpu.sync_copy(agg_sh.at[pl.ds(s * RPT, RPT)],
                    out_hbm.at[c, pl.ds(s * RPT, RPT)])


def _layer(h, src, dst, r, norm1, zeros, W, Wself, b):
    Wall = jnp.concatenate([W, Wself[None]], axis=0)
    proj = _proj_call(h, Wall)
    parts = _mp_call(proj.reshape(RP * N, D), src, dst, r, norm1, zeros)
    return _combine_call(parts, proj, b.reshape(1, D))


def kernel(h, edge_index, r, norm, W1, Wself1, b1, W2, Wself2, b2):
    norm1 = norm.reshape(E)
    src = edge_index[0]
    dst = edge_index[1]
    zeros = jnp.zeros((NP, D), jnp.float32)
    h1 = _layer(h, src, dst, r, norm1, zeros, W1, Wself1, b1)
    h2 = _layer(h1, src, dst, r, norm1, zeros, W2, Wself2, b2)
    return h2


# speedup vs baseline: 1.0380x; 1.0105x over previous
import functools

import jax
import jax.numpy as jnp
from jax import lax
from jax.experimental import pallas as pl
from jax.experimental.pallas import tpu as pltpu
from jax.experimental.pallas import tpu_sc as plsc

N = 10000
E = 320000
D = 128
R = 8
RP = R + 1

NC = 2
NS = 16
L = 16
NW = NC * NS
EPW = E // NW
C = 80
KCH = EPW // C
NBUF = 4
NP = 10240
RPT = NP // NS

BN = 2000

_GDN = lax.GatherDimensionNumbers(
    offset_dims=(), collapsed_slice_dims=(0,), start_index_map=(0,))


def _lane_bcast(vec, i):
    idx = jnp.full((L, 1), i, jnp.int32)
    return lax.gather(vec, idx, _GDN, (1,),
                      mode=lax.GatherScatterMode.PROMISE_IN_BOUNDS)


def _proj_body(h_ref, w_ref, out_ref):
    out_ref[0] = jnp.dot(h_ref[...], w_ref[0], preferred_element_type=jnp.float32)


_proj_call = pl.pallas_call(
    _proj_body,
    grid=(N // BN, RP),
    in_specs=[
        pl.BlockSpec((BN, D), lambda i, j: (i, 0)),
        pl.BlockSpec((1, D, D), lambda i, j: (j, 0, 0)),
    ],
    out_specs=pl.BlockSpec((1, BN, D), lambda i, j: (j, i, 0)),
    out_shape=jax.ShapeDtypeStruct((RP, N, D), jnp.float32),
)


def _combine_body(parts_ref, pself_ref, b_ref, out_ref):
    x = parts_ref[0] + parts_ref[1] + pself_ref[0] + b_ref[...]
    out_ref[...] = jnp.maximum(x, 0.0)


def _projcomb_body(parts_ref, pself_ref, b_ref, w_ref, out_ref):
    h2 = jnp.maximum(parts_ref[0] + parts_ref[1] + pself_ref[0] + b_ref[...],
                     0.0)
    out_ref[0] = jnp.dot(h2, w_ref[0], preferred_element_type=jnp.float32)


_projcomb_call = pl.pallas_call(
    _projcomb_body,
    grid=(N // BN, RP),
    in_specs=[
        pl.BlockSpec((2, BN, D), lambda i, j: (0, i, 0)),
        pl.BlockSpec((1, BN, D), lambda i, j: (R, i, 0)),
        pl.BlockSpec((1, D), lambda i, j: (0, 0)),
        pl.BlockSpec((1, D, D), lambda i, j: (j, 0, 0)),
    ],
    out_specs=pl.BlockSpec((1, BN, D), lambda i, j: (j, i, 0)),
    out_shape=jax.ShapeDtypeStruct((RP, N, D), jnp.float32),
)


_combine_call = pl.pallas_call(
    _combine_body,
    grid=(N // BN,),
    in_specs=[
        pl.BlockSpec((2, BN, D), lambda i: (0, i, 0)),
        pl.BlockSpec((1, BN, D), lambda i: (R, i, 0)),
        pl.BlockSpec((1, D), lambda i: (0, 0)),
    ],
    out_specs=pl.BlockSpec((BN, D), lambda i: (i, 0)),
    out_shape=jax.ShapeDtypeStruct((N, D), jnp.float32),
)


@functools.partial(
    pl.kernel,
    mesh=plsc.VectorSubcoreMesh(core_axis_name="c", subcore_axis_name="s"),
    out_type=jax.ShapeDtypeStruct((2, NP, D), jnp.float32),
    scratch_types=[
        [pltpu.VMEM((C,), jnp.int32) for _ in range(NBUF)],
        [pltpu.VMEM((C,), jnp.int32) for _ in range(NBUF)],
        [pltpu.VMEM((C,), jnp.int32) for _ in range(NBUF)],
        [pltpu.VMEM((C,), jnp.float32) for _ in range(NBUF)],
        [pltpu.VMEM((C,), jnp.int32) for _ in range(NBUF)],
        [pltpu.VMEM((C, D), jnp.float32) for _ in range(NBUF)],
        pltpu.VMEM_SHARED((NP, D), jnp.float32),
        [pltpu.SemaphoreType.DMA for _ in range(NBUF)],
        [pltpu.SemaphoreType.DMA for _ in range(NBUF)],
        [pltpu.SemaphoreType.DMA for _ in range(NBUF)],
    ],
)
def _mp_call(proj_hbm, src_hbm, dst_hbm, r_hbm, norm_hbm, zeros_hbm, out_hbm,
             srcc, relc, gidxc, normc, dstc, rows, agg_sh, sem_m, sem_g, sem_s):
    c = lax.axis_index("c")
    s = lax.axis_index("s")
    wid = s * NC + c
    ebase = wid * EPW

    pltpu.sync_copy(zeros_hbm.at[pl.ds(s * RPT, RPT)],
                    agg_sh.at[pl.ds(s * RPT, RPT)])
    plsc.subcore_barrier()

    def meta_start(k, b):
        base = ebase + k * C
        pltpu.async_copy(src_hbm.at[pl.ds(base, C)], srcc[b], sem_m[b])
        pltpu.async_copy(r_hbm.at[pl.ds(base, C)], relc[b], sem_m[b])
        pltpu.async_copy(norm_hbm.at[pl.ds(base, C)], normc[b], sem_m[b])
        pltpu.async_copy(dst_hbm.at[pl.ds(base, C)], dstc[b], sem_m[b])

    def meta_wait_build_gather(b):
        pltpu.make_async_copy(src_hbm.at[pl.ds(0, C)], srcc[b], sem_m[b]).wait()
        pltpu.make_async_copy(r_hbm.at[pl.ds(0, C)], relc[b], sem_m[b]).wait()
        pltpu.make_async_copy(norm_hbm.at[pl.ds(0, C)], normc[b], sem_m[b]).wait()
        pltpu.make_async_copy(dst_hbm.at[pl.ds(0, C)], dstc[b], sem_m[b]).wait()
        for t in range(C // L):
            sl = pl.ds(t * L, L)
            gidxc[b][sl] = relc[b][sl] * N + srcc[b][sl]
        pltpu.async_copy(proj_hbm.at[gidxc[b]], rows[b], sem_g[b])

    def gather_wait(b):
        pltpu.make_async_copy(proj_hbm.at[gidxc[b]], rows[b], sem_g[b]).wait()

    def scale(b):
        def _e16(t, cy):
            eb = t * L
            nv = normc[b][pl.ds(eb, L)]
            for i in range(L):
                nb = _lane_bcast(nv, i)
                for j in range(D // L):
                    sl = pl.ds(j * L, L)
                    rows[b][eb + i, sl] = rows[b][eb + i, sl] * nb
            return cy

        lax.fori_loop(0, C // L, _e16, 0)

    def scat_start(b):
        pltpu.async_copy(rows[b], agg_sh.at[dstc[b]], sem_s[b], add=True)

    def scat_wait(b):
        pltpu.make_async_copy(rows[b], agg_sh.at[dstc[b]], sem_s[b]).wait()

    meta_start(0, 0)
    meta_start(1, 1)
    meta_start(2, 2)
    meta_wait_build_gather(0)
    meta_wait_build_gather(1)

    def _group(g, carry):
        for b in range(NBUF):
            k = NBUF * g + b

            @pl.when(k < KCH)
            def _():
                @pl.when(k + 2 < KCH)
                def _():
                    meta_wait_build_gather((b + 2) % NBUF)

                gather_wait(b)
                scale(b)
                scat_start(b)

                @pl.when(k >= 1)
                def _():
                    scat_wait((b + 3) % NBUF)

                @pl.when(k + 3 < KCH)
                def _():
                    meta_start(k + 3, (b + 3) % NBUF)

        return carry

    lax.fori_loop(0, (KCH + NBUF - 1) // NBUF, _group, 0)
    scat_wait((KCH - 1) % NBUF)

    plsc.subcore_barrier()
    pltpu.sync_copy(agg_sh.at[pl.ds(s * RPT, RPT)],
                    out_hbm.at[c, pl.ds(s * RPT, RPT)])


def kernel(h, edge_index, r, norm, W1, Wself1, b1, W2, Wself2, b2):
    norm1 = norm.reshape(E)
    src = edge_index[0]
    dst = edge_index[1]
    zeros = jnp.zeros((NP, D), jnp.float32)
    Wall1 = jnp.concatenate([W1, Wself1[None]], axis=0)
    Wall2 = jnp.concatenate([W2, Wself2[None]], axis=0)
    proj1 = _proj_call(h, Wall1)
    parts1 = _mp_call(proj1.reshape(RP * N, D), src, dst, r, norm1, zeros)
    proj2 = _projcomb_call(parts1, proj1, b1.reshape(1, D), Wall2)
    parts2 = _mp_call(proj2.reshape(RP * N, D), src, dst, r, norm1, zeros)
    return _combine_call(parts2, proj2, b2.reshape(1, D))
